# baseline (speedup 1.0000x reference)
import jax
import jax.numpy as jnp
from jax import lax
from jax.experimental import pallas as pl
from jax.experimental.pallas import tpu as pltpu
from jax.experimental.pallas import tpu_sc as plsc

B, S, D_IN, D_OUT, L, E = 8, 2048, 128, 128, 4, 32768

NC, NS = 2, 16
BPC = B // NC
CHUNK = 128
EPT = E // NS
ROWS_PT = EPT // CHUNK
NROW = S // NS


def _mm_body(x_ref, w_ref, bias_ref, h_ref):
    x = x_ref[0]
    w = w_ref[...]
    h = jnp.dot(x, w, preferred_element_type=jnp.float32)
    h_ref[0] = h + bias_ref[...]


def _labelwise_transform(x, Wcat, bcat):
    return pl.pallas_call(
        _mm_body,
        grid=(B,),
        in_specs=[
            pl.BlockSpec((1, S, D_IN), lambda b: (b, 0, 0)),
            pl.BlockSpec((D_IN, L * D_OUT), lambda b: (0, 0)),
            pl.BlockSpec((1, L * D_OUT), lambda b: (0, 0)),
        ],
        out_specs=pl.BlockSpec((1, S, L * D_OUT), lambda b: (b, 0, 0)),
        out_shape=jax.ShapeDtypeStruct((B, S, L * D_OUT), jnp.float32),
    )(x, Wcat, bcat)


def _sc_body(h_ref, src_ref, tgt_ref, lbl_ref, zero_ref, out_ref,
             sv, lv, tv, gidx, rows0, rows1, rows2, rows3, rows4,
             obuf, acc, gsem0, gsem1, gsem2, ssem0, ssem1, ssem2):
    cid = lax.axis_index("c")
    sid = lax.axis_index("s")
    row0 = sid * ROWS_PT
    acc_row0 = sid * NROW
    rows = (rows0, rows1, rows2, rows3, rows4)
    gsems = (gsem0, gsem1, gsem2)
    ssems = (ssem0, ssem1, ssem2)

    pltpu.sync_copy(zero_ref, acc.at[pl.ds(acc_row0, NROW)])

    def relu_writeback(b):
        def relu_row(r, _):
            for i in range(D_OUT // 16):
                v = obuf[r, pl.ds(i * 16, 16)]
                obuf[r, pl.ds(i * 16, 16)] = jnp.maximum(v, 0.0)
            return 0

        lax.fori_loop(0, NROW, relu_row, 0)
        pltpu.sync_copy(obuf, out_ref.at[b, pl.ds(acc_row0, NROW)])

    def batch_body(bi, _):
        b = cid * BPC + bi

        pltpu.sync_copy(src_ref.at[b, pl.ds(row0, ROWS_PT)], sv)
        pltpu.sync_copy(tgt_ref.at[b, pl.ds(row0, ROWS_PT)], tv)
        pltpu.sync_copy(lbl_ref.at[b, pl.ds(row0, ROWS_PT)], lv)

        base = b * (S * L)
        for j in range(ROWS_PT):
            for i in range(CHUNK // 16):
                s16 = sv[j, pl.ds(i * 16, 16)]
                l16 = lv[j, pl.ds(i * 16, 16)]
                gidx[j, pl.ds(i * 16, 16)] = s16 * L + l16 + base

        plsc.subcore_barrier()

        gd = [None] * ROWS_PT
        sd = [None] * ROWS_PT
        for p in range(3):
            gd[p] = pltpu.async_copy(h_ref.at[gidx.at[p]], rows[p],
                                     gsems[p])

        @pl.when(bi > 0)
        def _():
            relu_writeback(b - 1)

        for j in range(ROWS_PT):
            gd[j].wait()
            sd[j] = pltpu.async_copy(rows[j % 5], acc.at[tv.at[j]],
                                     ssems[j % 3], add=True)
            if j >= 2:
                sd[j - 2].wait()
            if j + 3 < ROWS_PT:
                gd[j + 3] = pltpu.async_copy(
                    h_ref.at[gidx.at[j + 3]], rows[(j + 3) % 5],
                    gsems[j % 3])
        sd[ROWS_PT - 2].wait()
        sd[ROWS_PT - 1].wait()

        plsc.subcore_barrier()

        pltpu.sync_copy(acc.at[pl.ds(acc_row0, NROW)], obuf)
        pltpu.sync_copy(zero_ref, acc.at[pl.ds(acc_row0, NROW)])
        return 0

    lax.fori_loop(0, BPC, batch_body, 0)
    relu_writeback(cid * BPC + BPC - 1)


def _sc_scatter(h_flat, srcr, tgtr, lblr, zrow):
    mesh = plsc.VectorSubcoreMesh(core_axis_name="c", subcore_axis_name="s")
    k = pl.kernel(
        _sc_body,
        out_type=jax.ShapeDtypeStruct((B, S, D_OUT), jnp.float32),
        mesh=mesh,
        scratch_types=[
            pltpu.VMEM((ROWS_PT, CHUNK), jnp.int32),
            pltpu.VMEM((ROWS_PT, CHUNK), jnp.int32),
            pltpu.VMEM((ROWS_PT, CHUNK), jnp.int32),
            pltpu.VMEM((ROWS_PT, CHUNK), jnp.int32),
            pltpu.VMEM((CHUNK, D_OUT), jnp.float32),
            pltpu.VMEM((CHUNK, D_OUT), jnp.float32),
            pltpu.VMEM((CHUNK, D_OUT), jnp.float32),
            pltpu.VMEM((CHUNK, D_OUT), jnp.float32),
            pltpu.VMEM((CHUNK, D_OUT), jnp.float32),
            pltpu.VMEM((NROW, D_OUT), jnp.float32),
            pltpu.VMEM_SHARED((S, D_OUT), jnp.float32),
            pltpu.SemaphoreType.DMA,
            pltpu.SemaphoreType.DMA,
            pltpu.SemaphoreType.DMA,
            pltpu.SemaphoreType.DMA,
            pltpu.SemaphoreType.DMA,
            pltpu.SemaphoreType.DMA,
        ],
    )
    return k(h_flat, srcr, tgtr, lblr, zrow)


@jax.jit
def kernel(node_repr, edges, W, b):
    src = edges[..., 0].reshape(B, E // CHUNK, CHUNK)
    tgt = edges[..., 1].reshape(B, E // CHUNK, CHUNK)
    lbl = edges[..., 2].reshape(B, E // CHUNK, CHUNK)

    Wcat = W.transpose(2, 0, 1).reshape(D_IN, L * D_OUT)
    bcat = b.reshape(1, L * D_OUT)
    h = _labelwise_transform(node_repr, Wcat, bcat)
    h_flat = h.reshape(B * S * L, D_OUT)

    zrow = jnp.zeros((NROW, D_OUT), dtype=jnp.float32)
    return _sc_scatter(h_flat, src, tgt, lbl, zrow)


# speedup vs baseline: 1.1869x; 1.1869x over previous
import jax
import jax.numpy as jnp
from jax import lax
from jax.experimental import pallas as pl
from jax.experimental.pallas import tpu as pltpu
from jax.experimental.pallas import tpu_sc as plsc

B, S, D_IN, D_OUT, L, E = 8, 2048, 128, 128, 4, 32768

NC, NS = 2, 16
BPC = B // NC
CHUNK = 128
EPT = E // NS
ROWS_PT = EPT // CHUNK
NROW = S // NS


def _mm_body(x_ref, w_ref, bias_ref, h_ref):
    x = x_ref[0]
    w = w_ref[0]
    h = lax.dot_general(x, w, (((1,), (1,)), ((), ())),
                        preferred_element_type=jnp.float32)
    h_ref[0, 0] = h + bias_ref[pl.program_id(1)][None, :]


def _labelwise_transform(x, W, bias):
    return pl.pallas_call(
        _mm_body,
        grid=(B, L),
        in_specs=[
            pl.BlockSpec((1, S, D_IN), lambda b, l: (b, 0, 0)),
            pl.BlockSpec((1, D_OUT, D_IN), lambda b, l: (l, 0, 0)),
            pl.BlockSpec((L, D_OUT), lambda b, l: (0, 0)),
        ],
        out_specs=pl.BlockSpec((1, 1, S, D_OUT), lambda b, l: (b, l, 0, 0)),
        out_shape=jax.ShapeDtypeStruct((B, L, S, D_OUT), jnp.float32),
    )(x, W, bias)


def _sc_body(h_ref, src_ref, tgt_ref, lbl_ref, zero_ref, out_ref,
             sv, lv, tva, tvb, gidxa, gidxb, rows0, rows1, rows2, rows3,
             rows4, obuf, acc, gsem0, gsem1, gsem2, ssem0, ssem1, ssem2):
    cid = lax.axis_index("c")
    sid = lax.axis_index("s")
    row0 = sid * ROWS_PT
    acc_row0 = sid * NROW
    rows = (rows0, rows1, rows2, rows3, rows4)
    gsems = (gsem0, gsem1, gsem2)
    ssems = (ssem0, ssem1, ssem2)
    tvs = (tva, tvb)
    gidxs = (gidxa, gidxb)

    def stage_edges(b, tv):
        pltpu.sync_copy(src_ref.at[b, pl.ds(row0, ROWS_PT)], sv)
        pltpu.sync_copy(tgt_ref.at[b, pl.ds(row0, ROWS_PT)], tv)
        pltpu.sync_copy(lbl_ref.at[b, pl.ds(row0, ROWS_PT)], lv)

    def compute_gidx(b, gidx, j):
        base = b * (L * S)
        for i in range(CHUNK // 16):
            s16 = sv[j, pl.ds(i * 16, 16)]
            l16 = lv[j, pl.ds(i * 16, 16)]
            gidx[j, pl.ds(i * 16, 16)] = l16 * S + s16 + base

    def relu_writeback(b):
        def relu_row(r, _):
            for i in range(D_OUT // 16):
                v = obuf[r, pl.ds(i * 16, 16)]
                obuf[r, pl.ds(i * 16, 16)] = jnp.maximum(v, 0.0)
            return 0

        lax.fori_loop(0, NROW, relu_row, 0)
        pltpu.sync_copy(obuf, out_ref.at[b, pl.ds(acc_row0, NROW)])

    b0 = cid * BPC
    pltpu.sync_copy(zero_ref, acc.at[pl.ds(acc_row0, NROW)])
    stage_edges(b0, tvs[0])
    for j in range(ROWS_PT):
        compute_gidx(b0, gidxs[0], j)

    for bi in range(BPC):
        b = b0 + bi
        tv = tvs[bi % 2]
        gidx = gidxs[bi % 2]

        plsc.subcore_barrier()

        gd = [None] * ROWS_PT
        sd = [None] * ROWS_PT
        for p in range(3):
            gd[p] = pltpu.async_copy(h_ref.at[gidx.at[p]], rows[p],
                                     gsems[p])
        if bi > 0:
            relu_writeback(b - 1)

        for j in range(ROWS_PT):
            gd[j].wait()
            sd[j] = pltpu.async_copy(rows[j % 5], acc.at[tv.at[j]],
                                     ssems[j % 3], add=True)
            if j >= 2:
                sd[j - 2].wait()
            if j + 3 < ROWS_PT:
                gd[j + 3] = pltpu.async_copy(
                    h_ref.at[gidx.at[j + 3]], rows[(j + 3) % 5],
                    gsems[j % 3])
            if bi + 1 < BPC:
                if j == 2:
                    stage_edges(b + 1, tvs[(bi + 1) % 2])
                if 3 <= j < 3 + ROWS_PT // 2:
                    compute_gidx(b + 1, gidxs[(bi + 1) % 2],
                                 2 * (j - 3))
                    compute_gidx(b + 1, gidxs[(bi + 1) % 2],
                                 2 * (j - 3) + 1)
        sd[ROWS_PT - 2].wait()
        sd[ROWS_PT - 1].wait()

        plsc.subcore_barrier()

        pltpu.sync_copy(acc.at[pl.ds(acc_row0, NROW)], obuf)
        if bi + 1 < BPC:
            pltpu.sync_copy(zero_ref, acc.at[pl.ds(acc_row0, NROW)])

    relu_writeback(b0 + BPC - 1)


def _sc_scatter(h_flat, srcr, tgtr, lblr, zrow):
    mesh = plsc.VectorSubcoreMesh(core_axis_name="c", subcore_axis_name="s")
    k = pl.kernel(
        _sc_body,
        out_type=jax.ShapeDtypeStruct((B, S, D_OUT), jnp.float32),
        mesh=mesh,
        scratch_types=[
            pltpu.VMEM((ROWS_PT, CHUNK), jnp.int32),
            pltpu.VMEM((ROWS_PT, CHUNK), jnp.int32),
            pltpu.VMEM((ROWS_PT, CHUNK), jnp.int32),
            pltpu.VMEM((ROWS_PT, CHUNK), jnp.int32),
            pltpu.VMEM((ROWS_PT, CHUNK), jnp.int32),
            pltpu.VMEM((ROWS_PT, CHUNK), jnp.int32),
            pltpu.VMEM((CHUNK, D_OUT), jnp.float32),
            pltpu.VMEM((CHUNK, D_OUT), jnp.float32),
            pltpu.VMEM((CHUNK, D_OUT), jnp.float32),
            pltpu.VMEM((CHUNK, D_OUT), jnp.float32),
            pltpu.VMEM((CHUNK, D_OUT), jnp.float32),
            pltpu.VMEM((NROW, D_OUT), jnp.float32),
            pltpu.VMEM_SHARED((S, D_OUT), jnp.float32),
            pltpu.SemaphoreType.DMA,
            pltpu.SemaphoreType.DMA,
            pltpu.SemaphoreType.DMA,
            pltpu.SemaphoreType.DMA,
            pltpu.SemaphoreType.DMA,
            pltpu.SemaphoreType.DMA,
        ],
    )
    return k(h_flat, srcr, tgtr, lblr, zrow)


@jax.jit
def kernel(node_repr, edges, W, b):
    src = edges[..., 0].reshape(B, E // CHUNK, CHUNK)
    tgt = edges[..., 1].reshape(B, E // CHUNK, CHUNK)
    lbl = edges[..., 2].reshape(B, E // CHUNK, CHUNK)

    h = _labelwise_transform(node_repr, W, b)
    h_flat = h.reshape(B * L * S, D_OUT)

    zrow = jnp.zeros((NROW, D_OUT), dtype=jnp.float32)
    return _sc_scatter(h_flat, src, tgt, lbl, zrow)


# speedup vs baseline: 1.1918x; 1.0042x over previous
import jax
import jax.numpy as jnp
from jax import lax
from jax.experimental import pallas as pl
from jax.experimental.pallas import tpu as pltpu
from jax.experimental.pallas import tpu_sc as plsc

B, S, D_IN, D_OUT, L, E = 8, 2048, 128, 128, 4, 32768

NC, NS = 2, 16
BPC = B // NC
CHUNK = 128
EPT = E // NS
ROWS_PT = EPT // CHUNK
NROW = S // NS


def _mm_body(x_ref, w_ref, bias_ref, h_ref):
    x = x_ref[0].astype(jnp.bfloat16)
    w = w_ref[0].astype(jnp.bfloat16)
    h = lax.dot_general(x, w, (((1,), (1,)), ((), ())),
                        preferred_element_type=jnp.float32)
    h_ref[0, 0] = h + bias_ref[pl.program_id(1)][None, :]


def _labelwise_transform(x, W, bias):
    return pl.pallas_call(
        _mm_body,
        grid=(B, L),
        in_specs=[
            pl.BlockSpec((1, S, D_IN), lambda b, l: (b, 0, 0)),
            pl.BlockSpec((1, D_OUT, D_IN), lambda b, l: (l, 0, 0)),
            pl.BlockSpec((L, D_OUT), lambda b, l: (0, 0)),
        ],
        out_specs=pl.BlockSpec((1, 1, S, D_OUT), lambda b, l: (b, l, 0, 0)),
        out_shape=jax.ShapeDtypeStruct((B, L, S, D_OUT), jnp.float32),
    )(x, W, bias)


def _sc_body(h_ref, src_ref, tgt_ref, lbl_ref, zero_ref, out_ref,
             sv, lv, tva, tvb, gidxa, gidxb, rows0, rows1, rows2, rows3,
             rows4, obuf, acc, gsem0, gsem1, gsem2, ssem0, ssem1, ssem2):
    cid = lax.axis_index("c")
    sid = lax.axis_index("s")
    row0 = sid * ROWS_PT
    acc_row0 = sid * NROW
    rows = (rows0, rows1, rows2, rows3, rows4)
    gsems = (gsem0, gsem1, gsem2)
    ssems = (ssem0, ssem1, ssem2)
    tvs = (tva, tvb)
    gidxs = (gidxa, gidxb)

    def stage_edges(b, tv):
        pltpu.sync_copy(src_ref.at[b, pl.ds(row0, ROWS_PT)], sv)
        pltpu.sync_copy(tgt_ref.at[b, pl.ds(row0, ROWS_PT)], tv)
        pltpu.sync_copy(lbl_ref.at[b, pl.ds(row0, ROWS_PT)], lv)

    def compute_gidx(b, gidx, j):
        base = b * (L * S)
        for i in range(CHUNK // 16):
            s16 = sv[j, pl.ds(i * 16, 16)]
            l16 = lv[j, pl.ds(i * 16, 16)]
            gidx[j, pl.ds(i * 16, 16)] = l16 * S + s16 + base

    def relu_writeback(b):
        def relu_row(r, _):
            for i in range(D_OUT // 16):
                v = obuf[r, pl.ds(i * 16, 16)]
                obuf[r, pl.ds(i * 16, 16)] = jnp.maximum(v, 0.0)
            return 0

        lax.fori_loop(0, NROW, relu_row, 0)
        pltpu.sync_copy(obuf, out_ref.at[b, pl.ds(acc_row0, NROW)])

    b0 = cid * BPC
    pltpu.sync_copy(zero_ref, acc.at[pl.ds(acc_row0, NROW)])
    stage_edges(b0, tvs[0])
    for j in range(ROWS_PT):
        compute_gidx(b0, gidxs[0], j)

    for bi in range(BPC):
        b = b0 + bi
        tv = tvs[bi % 2]
        gidx = gidxs[bi % 2]

        plsc.subcore_barrier()

        gd = [None] * ROWS_PT
        sd = [None] * ROWS_PT
        for p in range(3):
            gd[p] = pltpu.async_copy(h_ref.at[gidx.at[p]], rows[p],
                                     gsems[p])
        if bi > 0:
            relu_writeback(b - 1)

        for j in range(ROWS_PT):
            gd[j].wait()
            sd[j] = pltpu.async_copy(rows[j % 5], acc.at[tv.at[j]],
                                     ssems[j % 3], add=True)
            if j >= 2:
                sd[j - 2].wait()
            if j + 3 < ROWS_PT:
                gd[j + 3] = pltpu.async_copy(
                    h_ref.at[gidx.at[j + 3]], rows[(j + 3) % 5],
                    gsems[j % 3])
            if bi + 1 < BPC:
                if j == 2:
                    stage_edges(b + 1, tvs[(bi + 1) % 2])
                if 3 <= j < 3 + ROWS_PT // 2:
                    compute_gidx(b + 1, gidxs[(bi + 1) % 2],
                                 2 * (j - 3))
                    compute_gidx(b + 1, gidxs[(bi + 1) % 2],
                                 2 * (j - 3) + 1)
        sd[ROWS_PT - 2].wait()
        sd[ROWS_PT - 1].wait()

        plsc.subcore_barrier()

        pltpu.sync_copy(acc.at[pl.ds(acc_row0, NROW)], obuf)
        if bi + 1 < BPC:
            pltpu.sync_copy(zero_ref, acc.at[pl.ds(acc_row0, NROW)])

    relu_writeback(b0 + BPC - 1)


def _sc_scatter(h_flat, srcr, tgtr, lblr, zrow):
    mesh = plsc.VectorSubcoreMesh(core_axis_name="c", subcore_axis_name="s")
    k = pl.kernel(
        _sc_body,
        out_type=jax.ShapeDtypeStruct((B, S, D_OUT), jnp.float32),
        mesh=mesh,
        scratch_types=[
            pltpu.VMEM((ROWS_PT, CHUNK), jnp.int32),
            pltpu.VMEM((ROWS_PT, CHUNK), jnp.int32),
            pltpu.VMEM((ROWS_PT, CHUNK), jnp.int32),
            pltpu.VMEM((ROWS_PT, CHUNK), jnp.int32),
            pltpu.VMEM((ROWS_PT, CHUNK), jnp.int32),
            pltpu.VMEM((ROWS_PT, CHUNK), jnp.int32),
            pltpu.VMEM((CHUNK, D_OUT), jnp.float32),
            pltpu.VMEM((CHUNK, D_OUT), jnp.float32),
            pltpu.VMEM((CHUNK, D_OUT), jnp.float32),
            pltpu.VMEM((CHUNK, D_OUT), jnp.float32),
            pltpu.VMEM((CHUNK, D_OUT), jnp.float32),
            pltpu.VMEM((NROW, D_OUT), jnp.float32),
            pltpu.VMEM_SHARED((S, D_OUT), jnp.float32),
            pltpu.SemaphoreType.DMA,
            pltpu.SemaphoreType.DMA,
            pltpu.SemaphoreType.DMA,
            pltpu.SemaphoreType.DMA,
            pltpu.SemaphoreType.DMA,
            pltpu.SemaphoreType.DMA,
        ],
    )
    return k(h_flat, srcr, tgtr, lblr, zrow)


@jax.jit
def kernel(node_repr, edges, W, b):
    src = edges[..., 0].reshape(B, E // CHUNK, CHUNK)
    tgt = edges[..., 1].reshape(B, E // CHUNK, CHUNK)
    lbl = edges[..., 2].reshape(B, E // CHUNK, CHUNK)

    h = _labelwise_transform(node_repr, W, b)
    h_flat = h.reshape(B * L * S, D_OUT)

    zrow = jnp.zeros((NROW, D_OUT), dtype=jnp.float32)
    return _sc_scatter(h_flat, src, tgt, lbl, zrow)
